# baseline (speedup 1.0000x reference)
import functools

import jax
import jax.numpy as jnp
from jax import lax
from jax.experimental import pallas as pl
from jax.experimental.pallas import tpu as pltpu
from jax.experimental.pallas import tpu_sc as plsc

_INFO = plsc.get_sparse_core_info()
_NC = _INFO.num_cores
_NS = _INFO.num_subcores
_NW = _NC * _NS
_L = _INFO.num_lanes


def _gather_call(indices, table, B, D):
    G = 128 // D
    b_per_w = B // _NW
    o_per_w = b_per_w // G

    mesh = plsc.VectorSubcoreMesh(core_axis_name="c", subcore_axis_name="s")

    @functools.partial(
        pl.kernel,
        mesh=mesh,
        out_type=jax.ShapeDtypeStruct((B // G, 128), jnp.float32),
        scratch_types=[
            pltpu.VMEM((b_per_w,), jnp.int32),
            pltpu.VMEM((b_per_w,), jnp.int32),
            pltpu.VMEM((b_per_w, 128), jnp.float32),
            pltpu.VMEM((o_per_w, 128), jnp.float32),
            pltpu.SemaphoreType.DMA,
        ],
        compiler_params=pltpu.CompilerParams(needs_layout_passes=False),
    )
    def gather_kernel(idx_hbm, table_hbm, out_hbm,
                      idx_v, grp_v, rows_v, out_v, sem):
        wid = lax.axis_index("s") * _NC + lax.axis_index("c")
        base = wid * b_per_w

        pltpu.sync_copy(idx_hbm.at[pl.ds(base, b_per_w)], idx_v)

        def grp_body(i, _):
            grp_v[pl.ds(i * _L, _L)] = lax.shift_right_logical(
                idx_v[pl.ds(i * _L, _L)], 2)
            return _
        lax.fori_loop(0, b_per_w // _L, grp_body, None)

        pltpu.async_copy(table_hbm.at[grp_v], rows_v, sem).wait()

        lanes = lax.iota(jnp.int32, _L)

        def sel_body(blk, _):
            b0 = blk * _L
            bvec = b0 + lanes
            ivec = idx_v[pl.ds(b0, _L)]
            src0 = lax.shift_left(ivec & 3, 5)
            dst0 = lax.shift_left(bvec, 5)
            for j in range(D):
                val = plsc.load_gather(rows_v, [bvec, src0 + j])
                flat = dst0 + j
                plsc.store_scatter(
                    out_v,
                    [lax.shift_right_logical(flat, 7), flat & 127],
                    val,
                )
            return _
        lax.fori_loop(0, b_per_w // _L, sel_body, None)

        pltpu.sync_copy(out_v, out_hbm.at[pl.ds(wid * o_per_w, o_per_w)])

    return gather_kernel(indices, table)


def kernel(indices, mean, logstd):
    del logstd
    V, D = mean.shape
    B, = indices.shape
    G = 128 // D
    table = mean.reshape(V // G, D * G)
    out = _gather_call(indices.astype(jnp.int32), table, B, D)
    return out.reshape(B, D)


# speedup vs baseline: 1.3622x; 1.3622x over previous
import functools

import jax
import jax.numpy as jnp
from jax import lax
from jax.experimental import pallas as pl
from jax.experimental.pallas import tpu as pltpu
from jax.experimental.pallas import tpu_sc as plsc

_INFO = plsc.get_sparse_core_info()
_NC = _INFO.num_cores
_NS = _INFO.num_subcores
_NW = _NC * _NS
_L = _INFO.num_lanes


def _tc_relayout(mean_t, V, D):
    cols = 12544
    gout = cols // 4
    grid = (V + cols - 1) // cols

    def body(in_ref, out_ref):
        y = jnp.transpose(in_ref[...])
        y3 = y.reshape(gout, 4, D)
        for r in range(4):
            out_ref[:, r * D:(r + 1) * D] = y3[:, r, :]

    return pl.pallas_call(
        body,
        grid=(grid,),
        in_specs=[pl.BlockSpec((D, cols), lambda i: (0, i))],
        out_specs=pl.BlockSpec((gout, 128), lambda i: (i, 0)),
        out_shape=jax.ShapeDtypeStruct((V // 4, 128), jnp.float32),
    )(mean_t)


def _sc_gather(indices, table, B, D):
    b_per_w = B // _NW
    o_per_w = b_per_w * D // 128

    mesh = plsc.VectorSubcoreMesh(core_axis_name="c", subcore_axis_name="s")

    @functools.partial(
        pl.kernel,
        mesh=mesh,
        out_type=jax.ShapeDtypeStruct((B * D // 128, 128), jnp.float32),
        scratch_types=[
            pltpu.VMEM((b_per_w,), jnp.int32),
            pltpu.VMEM((b_per_w,), jnp.int32),
            pltpu.VMEM((b_per_w, 128), jnp.float32),
            pltpu.VMEM((o_per_w, 128), jnp.float32),
            pltpu.SemaphoreType.DMA,
        ],
        compiler_params=pltpu.CompilerParams(needs_layout_passes=False),
    )
    def gather_kernel(idx_hbm, table_hbm, out_hbm,
                      idx_v, grp_v, rows_v, out_v, sem):
        wid = lax.axis_index("s") * _NC + lax.axis_index("c")
        base = wid * b_per_w

        pltpu.sync_copy(idx_hbm.at[pl.ds(base, b_per_w)], idx_v)

        def grp_body(i, _):
            grp_v[pl.ds(i * _L, _L)] = lax.shift_right_logical(
                idx_v[pl.ds(i * _L, _L)], 2)
            return _
        lax.fori_loop(0, b_per_w // _L, grp_body, None)

        pltpu.async_copy(table_hbm.at[grp_v], rows_v, sem).wait()

        lanes = lax.iota(jnp.int32, _L)

        def sel_body(blk, _):
            b0 = blk * _L
            bvec = b0 + lanes
            ivec = idx_v[pl.ds(b0, _L)]
            src0 = lax.shift_left(ivec & 3, 5)
            dst0 = lax.shift_left(bvec, 5)
            for j in range(D):
                val = plsc.load_gather(rows_v, [bvec, src0 + j])
                flat = dst0 + j
                plsc.store_scatter(
                    out_v,
                    [lax.shift_right_logical(flat, 7), flat & 127],
                    val,
                )
            return _
        lax.fori_loop(0, b_per_w // _L, sel_body, None)

        pltpu.sync_copy(out_v, out_hbm.at[pl.ds(wid * o_per_w, o_per_w)])

    return gather_kernel(indices, table)


def kernel(indices, mean, logstd):
    del logstd
    V, D = mean.shape
    B, = indices.shape
    table = _tc_relayout(mean.T, V, D)
    out = _sc_gather(indices.astype(jnp.int32), table, B, D)
    return out.reshape(B, D)


# speedup vs baseline: 1.8356x; 1.3475x over previous
import functools

import jax
import jax.numpy as jnp
from jax import lax
from jax.experimental import pallas as pl
from jax.experimental.pallas import tpu as pltpu
from jax.experimental.pallas import tpu_sc as plsc

_INFO = plsc.get_sparse_core_info()
_NC = _INFO.num_cores
_NS = _INFO.num_subcores
_NW = _NC * _NS
_L = _INFO.num_lanes


def _tc_relayout(mean_t, V, D):
    cols = 16384
    gout = cols // 4
    grid = (V + cols - 1) // cols

    def body(in_ref, out_ref):
        y = jnp.transpose(in_ref[...])
        for s in range(4):
            out_ref[:, s * D:(s + 1) * D] = y[s * gout:(s + 1) * gout, :]

    return pl.pallas_call(
        body,
        grid=(grid,),
        in_specs=[pl.BlockSpec((D, cols), lambda i: (0, i))],
        out_specs=pl.BlockSpec((gout, 128), lambda i: (i, 0)),
        out_shape=jax.ShapeDtypeStruct((grid * gout, 128), jnp.float32),
    )(mean_t)


def _sc_gather(indices, table, B, D):
    b_per_w = B // _NW
    o_per_w = b_per_w * D // 128

    mesh = plsc.VectorSubcoreMesh(core_axis_name="c", subcore_axis_name="s")

    @functools.partial(
        pl.kernel,
        mesh=mesh,
        out_type=jax.ShapeDtypeStruct((B * D // 128, 128), jnp.float32),
        scratch_types=[
            pltpu.VMEM((b_per_w,), jnp.int32),
            pltpu.VMEM((b_per_w,), jnp.int32),
            pltpu.VMEM((b_per_w, 128), jnp.float32),
            pltpu.VMEM((o_per_w, 128), jnp.float32),
            pltpu.SemaphoreType.DMA,
        ],
        compiler_params=pltpu.CompilerParams(needs_layout_passes=False),
    )
    def gather_kernel(idx_hbm, table_hbm, out_hbm,
                      idx_v, grp_v, rows_v, out_v, sem):
        wid = lax.axis_index("s") * _NC + lax.axis_index("c")
        base = wid * b_per_w

        pltpu.sync_copy(idx_hbm.at[pl.ds(base, b_per_w)], idx_v)

        def grp_body(i, _):
            ivec = idx_v[pl.ds(i * _L, _L)]
            grp_v[pl.ds(i * _L, _L)] = lax.shift_left(
                lax.shift_right_logical(ivec, 14), 12) + (ivec & 4095)
            return _
        lax.fori_loop(0, b_per_w // _L, grp_body, None)

        pltpu.async_copy(table_hbm.at[grp_v], rows_v, sem).wait()

        lanes = lax.iota(jnp.int32, _L)

        def sel_body(blk, _):
            b0 = blk * _L
            bvec = b0 + lanes
            ivec = idx_v[pl.ds(b0, _L)]
            src0 = lax.shift_left(
                lax.shift_right_logical(ivec, 12) & 3, 5)
            dst0 = lax.shift_left(bvec, 5)
            for j in range(D):
                val = plsc.load_gather(rows_v, [bvec, src0 + j])
                flat = dst0 + j
                plsc.store_scatter(
                    out_v,
                    [lax.shift_right_logical(flat, 7), flat & 127],
                    val,
                )
            return _
        lax.fori_loop(0, b_per_w // _L, sel_body, None)

        pltpu.sync_copy(out_v, out_hbm.at[pl.ds(wid * o_per_w, o_per_w)])

    return gather_kernel(indices, table)


def kernel(indices, mean, logstd):
    del logstd
    V, D = mean.shape
    B, = indices.shape
    table = _tc_relayout(mean.T, V, D)
    out = _sc_gather(indices.astype(jnp.int32), table, B, D)
    return out.reshape(B, D)


# speedup vs baseline: 1.8422x; 1.0036x over previous
import functools

import jax
import jax.numpy as jnp
from jax import lax
from jax.experimental import pallas as pl
from jax.experimental.pallas import tpu as pltpu
from jax.experimental.pallas import tpu_sc as plsc

_INFO = plsc.get_sparse_core_info()
_NC = _INFO.num_cores
_NS = _INFO.num_subcores
_NW = _NC * _NS
_L = _INFO.num_lanes


def _tc_relayout(mean_t, V, D):
    cols = 32768
    gout = cols // 4
    grid = (V + cols - 1) // cols

    def body(in_ref, out_ref):
        y = jnp.transpose(in_ref[...])
        for s in range(4):
            out_ref[:, s * D:(s + 1) * D] = y[s * gout:(s + 1) * gout, :]

    return pl.pallas_call(
        body,
        grid=(grid,),
        in_specs=[pl.BlockSpec((D, cols), lambda i: (0, i))],
        out_specs=pl.BlockSpec((gout, 128), lambda i: (i, 0)),
        out_shape=jax.ShapeDtypeStruct((grid * gout, 128), jnp.float32),
    )(mean_t)


def _sc_gather(indices, table, B, D):
    b_per_w = B // _NW
    o_per_w = b_per_w * D // 128

    mesh = plsc.VectorSubcoreMesh(core_axis_name="c", subcore_axis_name="s")

    @functools.partial(
        pl.kernel,
        mesh=mesh,
        out_type=jax.ShapeDtypeStruct((B * D // 128, 128), jnp.float32),
        scratch_types=[
            pltpu.VMEM((b_per_w,), jnp.int32),
            pltpu.VMEM((b_per_w,), jnp.int32),
            pltpu.VMEM((b_per_w, 128), jnp.float32),
            pltpu.VMEM((o_per_w, 128), jnp.float32),
            pltpu.SemaphoreType.DMA,
        ],
        compiler_params=pltpu.CompilerParams(needs_layout_passes=False),
    )
    def gather_kernel(idx_hbm, table_hbm, out_hbm,
                      idx_v, grp_v, rows_v, out_v, sem):
        wid = lax.axis_index("s") * _NC + lax.axis_index("c")
        base = wid * b_per_w

        pltpu.sync_copy(idx_hbm.at[pl.ds(base, b_per_w)], idx_v)

        def grp_body(i, _):
            ivec = idx_v[pl.ds(i * _L, _L)]
            grp_v[pl.ds(i * _L, _L)] = lax.shift_left(
                lax.shift_right_logical(ivec, 15), 13) + (ivec & 8191)
            return _
        lax.fori_loop(0, b_per_w // _L, grp_body, None)

        pltpu.async_copy(table_hbm.at[grp_v], rows_v, sem).wait()

        lanes = lax.iota(jnp.int32, _L)

        def sel_body(blk, _):
            b0 = blk * _L
            bvec = b0 + lanes
            ivec = idx_v[pl.ds(b0, _L)]
            src0 = lax.shift_left(
                lax.shift_right_logical(ivec, 13) & 3, 5)
            dst0 = lax.shift_left(bvec, 5)
            for j in range(D):
                val = plsc.load_gather(rows_v, [bvec, src0 + j])
                flat = dst0 + j
                plsc.store_scatter(
                    out_v,
                    [lax.shift_right_logical(flat, 7), flat & 127],
                    val,
                )
            return _
        lax.fori_loop(0, b_per_w // _L, sel_body, None)

        pltpu.sync_copy(out_v, out_hbm.at[pl.ds(wid * o_per_w, o_per_w)])

    return gather_kernel(indices, table)


def kernel(indices, mean, logstd):
    del logstd
    V, D = mean.shape
    B, = indices.shape
    table = _tc_relayout(mean.T, V, D)
    out = _sc_gather(indices.astype(jnp.int32), table, B, D)
    return out.reshape(B, D)
